# baseline (speedup 1.0000x reference)
import functools

import jax
import jax.numpy as jnp
from jax import lax
from jax.experimental import pallas as pl
from jax.experimental.pallas import tpu as pltpu
from jax.experimental.pallas import tpu_sc as plsc

_N = 10000
_E = 320000
_D = 128

_NC = 2
_NS = 16
_NW = _NC * _NS
_EPW = _E // _NW
_CH = 50
_NCHUNK = _EPW // _CH
_K = 4
_RPT = _N // _NS

_R = 1000
_GRID = _N // _R

_mesh = plsc.VectorSubcoreMesh(core_axis_name="c", subcore_axis_name="s")


_DW = 8


@functools.partial(
    pl.kernel,
    out_type=jax.ShapeDtypeStruct((_NC, _N, _DW), jnp.float32),
    mesh=_mesh,
    scratch_types=[
        pltpu.VMEM((_NCHUNK, _CH), jnp.int32),
        pltpu.VMEM((_CH, _DW), jnp.float32),
        pltpu.VMEM_SHARED((_N, _DW), jnp.float32),
    ],
    compiler_params=pltpu.CompilerParams(use_tc_tiling_on_sc=False),
)
def _deg_kernel(dst_hbm, ones_hbm, out_hbm, dstbuf, onesbuf, acc):
    c = lax.axis_index("c")
    s = lax.axis_index("s")
    w = s * _NC + c
    pltpu.sync_copy(dst_hbm.at[w], dstbuf)
    pltpu.sync_copy(ones_hbm.at[pl.ds(0, _CH)], onesbuf)
    pltpu.sync_copy(ones_hbm.at[pl.ds(s * _RPT, _RPT)],
                    acc.at[pl.ds(s * _RPT, _RPT)])
    plsc.subcore_barrier()

    def body(j, carry):
        pltpu.sync_copy(onesbuf, acc.at[dstbuf.at[j]], add=True)
        return carry

    lax.fori_loop(0, _NCHUNK, body, 0)
    plsc.subcore_barrier()
    pltpu.sync_copy(acc.at[pl.ds(s * _RPT, _RPT)],
                    out_hbm.at[c].at[pl.ds(s * _RPT, _RPT)])


@functools.partial(
    pl.kernel,
    out_type=jax.ShapeDtypeStruct((_NC, _N, _D), jnp.float32),
    mesh=_mesh,
    scratch_types=[
        pltpu.VMEM((_NCHUNK, _CH), jnp.int32),
        pltpu.VMEM((_NCHUNK, _CH), jnp.int32),
        pltpu.VMEM((_K, _CH, _D), jnp.float32),
        pltpu.VMEM_SHARED((_N, _D), jnp.float32),
        [pltpu.SemaphoreType.DMA] * _K,
        [pltpu.SemaphoreType.DMA] * _K,
    ],
    compiler_params=pltpu.CompilerParams(use_tc_tiling_on_sc=False),
)
def _agg_kernel(h_hbm, src_hbm, dst_hbm, out_hbm, srcbuf, dstbuf, rows, acc,
                gsems, ssems):
    c = lax.axis_index("c")
    s = lax.axis_index("s")
    w = s * _NC + c
    cp0 = pltpu.async_copy(src_hbm.at[w], srcbuf, gsems[0])
    cp1 = pltpu.async_copy(dst_hbm.at[w], dstbuf, gsems[1])
    cp2 = pltpu.async_copy(h_hbm.at[pl.ds(s * _RPT, _RPT)],
                           acc.at[pl.ds(s * _RPT, _RPT)], gsems[2])
    cp0.wait()
    cp1.wait()
    cp2.wait()
    plsc.subcore_barrier()

    def fire_g(chunk, b):
        pltpu.async_copy(h_hbm.at[srcbuf.at[chunk]], rows.at[b], gsems[b])

    def fire_s(chunk, b):
        pltpu.async_copy(rows.at[b], acc.at[dstbuf.at[chunk]], ssems[b],
                         add=True)

    def wait_g(b):
        pltpu.make_async_copy(h_hbm.at[srcbuf.at[0]], rows.at[b],
                              gsems[b]).wait()

    def wait_s(b):
        pltpu.make_async_copy(rows.at[b], acc.at[dstbuf.at[0]],
                              ssems[b]).wait()

    def group(g, carry):
        base = g * _K
        for b in range(_K):
            fire_g(base + b, b)
        for b in range(_K):
            wait_g(b)
        for b in range(_K):
            fire_s(base + b, b)
        for b in range(_K):
            wait_s(b)
        return carry

    lax.fori_loop(0, _NCHUNK // _K, group, 0)
    plsc.subcore_barrier()
    pltpu.sync_copy(acc.at[pl.ds(s * _RPT, _RPT)],
                    out_hbm.at[c].at[pl.ds(s * _RPT, _RPT)])


def _h1_body(degp_ref, x_ref, w1_ref, h1_ref, dinv_ref):
    deg = degp_ref[0, :, 0:1] + degp_ref[1, :, 0:1] - 1.0
    dinv = lax.rsqrt(deg)
    h = jnp.dot(x_ref[...], w1_ref[...], preferred_element_type=jnp.float32)
    h1_ref[...] = h * dinv
    dinv_ref[...] = dinv


def _l2_body(p_ref, h1_ref, dinv_ref, b1_ref, w2_ref, h2_ref):
    agg = p_ref[0] + p_ref[1] - h1_ref[...]
    t = jnp.maximum(agg * dinv_ref[...] + b1_ref[...], 0.0)
    h2 = jnp.dot(t, w2_ref[...], preferred_element_type=jnp.float32)
    h2_ref[...] = h2 * dinv_ref[...]


def _out_body(p_ref, h2_ref, dinv_ref, b2_ref, o_ref):
    agg = p_ref[0] + p_ref[1] - h2_ref[...]
    o = agg * dinv_ref[...] + b2_ref[...]
    m = jnp.max(o, axis=1, keepdims=True)
    lse = jnp.log(jnp.sum(jnp.exp(o - m), axis=1, keepdims=True))
    o_ref[...] = o - m - lse


def _row_spec(cols):
    return pl.BlockSpec((_R, cols), lambda i: (i, 0))


def _full_spec(shape):
    return pl.BlockSpec(shape, lambda i: tuple(0 for _ in shape))


def kernel(x, edge_index, W1, b1, W2, b2):
    src = edge_index[0].reshape(_NW, _NCHUNK, _CH)
    dst = edge_index[1].reshape(_NW, _NCHUNK, _CH)
    ones8 = jnp.ones((_N, _DW), jnp.float32)

    deg_parts = _deg_kernel(dst, ones8)

    h1, dinv = pl.pallas_call(
        _h1_body,
        grid=(_GRID,),
        in_specs=[pl.BlockSpec((_NC, _R, _DW), lambda i: (0, i, 0)),
                  _row_spec(_D), _full_spec((_D, _D))],
        out_specs=[_row_spec(_D), _row_spec(1)],
        out_shape=[
            jax.ShapeDtypeStruct((_N, _D), jnp.float32),
            jax.ShapeDtypeStruct((_N, 1), jnp.float32),
        ],
    )(deg_parts, x, W1)

    p1 = _agg_kernel(h1, src, dst)

    h2 = pl.pallas_call(
        _l2_body,
        grid=(_GRID,),
        in_specs=[
            pl.BlockSpec((_NC, _R, _D), lambda i: (0, i, 0)),
            _row_spec(_D), _row_spec(1), _full_spec((1, _D)),
            _full_spec((_D, _D)),
        ],
        out_specs=_row_spec(_D),
        out_shape=jax.ShapeDtypeStruct((_N, _D), jnp.float32),
    )(p1, h1, dinv, b1.reshape(1, _D), W2)

    p2 = _agg_kernel(h2, src, dst)

    out = pl.pallas_call(
        _out_body,
        grid=(_GRID,),
        in_specs=[
            pl.BlockSpec((_NC, _R, _D), lambda i: (0, i, 0)),
            _row_spec(_D), _row_spec(1), _full_spec((1, _D)),
        ],
        out_specs=_row_spec(_D),
        out_shape=jax.ShapeDtypeStruct((_N, _D), jnp.float32),
    )(p2, h2, dinv, b2.reshape(1, _D))

    return out


# speedup vs baseline: 1.0800x; 1.0800x over previous
import functools

import jax
import jax.numpy as jnp
from jax import lax
from jax.experimental import pallas as pl
from jax.experimental.pallas import tpu as pltpu
from jax.experimental.pallas import tpu_sc as plsc

_N = 10000
_E = 320000
_D = 128

_NC = 2
_NS = 16
_NW = _NC * _NS
_EPW = _E // _NW
_CH = 80
_NCHUNK = _EPW // _CH
_K = 2
_RPT = _N // _NS

_R = 1000
_GRID = _N // _R

_mesh = plsc.VectorSubcoreMesh(core_axis_name="c", subcore_axis_name="s")


_DW = 8


@functools.partial(
    pl.kernel,
    out_type=jax.ShapeDtypeStruct((_NC, _N, _DW), jnp.float32),
    mesh=_mesh,
    scratch_types=[
        pltpu.VMEM((_NCHUNK, _CH), jnp.int32),
        pltpu.VMEM((_CH, _DW), jnp.float32),
        pltpu.VMEM_SHARED((_N, _DW), jnp.float32),
    ],
    compiler_params=pltpu.CompilerParams(use_tc_tiling_on_sc=False),
)
def _deg_kernel(dst_hbm, ones_hbm, out_hbm, dstbuf, onesbuf, acc):
    c = lax.axis_index("c")
    s = lax.axis_index("s")
    w = s * _NC + c
    pltpu.sync_copy(dst_hbm.at[w], dstbuf)
    pltpu.sync_copy(ones_hbm.at[pl.ds(0, _CH)], onesbuf)
    pltpu.sync_copy(ones_hbm.at[pl.ds(s * _RPT, _RPT)],
                    acc.at[pl.ds(s * _RPT, _RPT)])
    plsc.subcore_barrier()

    def body(j, carry):
        pltpu.sync_copy(onesbuf, acc.at[dstbuf.at[j]], add=True)
        return carry

    lax.fori_loop(0, _NCHUNK, body, 0)
    plsc.subcore_barrier()
    pltpu.sync_copy(acc.at[pl.ds(s * _RPT, _RPT)],
                    out_hbm.at[c].at[pl.ds(s * _RPT, _RPT)])


@functools.partial(
    pl.kernel,
    out_type=jax.ShapeDtypeStruct((_NC, _N, _D), jnp.float32),
    mesh=_mesh,
    scratch_types=[
        pltpu.VMEM((_NCHUNK, _CH), jnp.int32),
        pltpu.VMEM((_NCHUNK, _CH), jnp.int32),
        pltpu.VMEM((_K, _CH, _D), jnp.float32),
        pltpu.VMEM_SHARED((_N, _D), jnp.float32),
        pltpu.SemaphoreType.DMA,
    ],
    compiler_params=pltpu.CompilerParams(use_tc_tiling_on_sc=False),
)
def _agg_kernel(h_hbm, src_hbm, dst_hbm, out_hbm, srcbuf, dstbuf, rows, acc,
                gsem):
    c = lax.axis_index("c")
    s = lax.axis_index("s")
    w = s * _NC + c
    cp0 = pltpu.async_copy(src_hbm.at[w], srcbuf, gsem)
    cp1 = pltpu.async_copy(dst_hbm.at[w], dstbuf, gsem)
    cp2 = pltpu.async_copy(h_hbm.at[pl.ds(s * _RPT, _RPT)],
                           acc.at[pl.ds(s * _RPT, _RPT)], gsem)
    cp0.wait()
    cp1.wait()
    cp2.wait()
    plsc.subcore_barrier()

    def group(g, carry):
        base = g * _K
        copies = [
            pltpu.async_copy(h_hbm.at[srcbuf.at[base + b]], rows.at[b], gsem)
            for b in range(_K)
        ]
        for b in range(_K):
            copies[b].wait()
            pltpu.sync_copy(rows.at[b], acc.at[dstbuf.at[base + b]], add=True)
        return carry

    lax.fori_loop(0, _NCHUNK // _K, group, 0)
    for r in range((_NCHUNK // _K) * _K, _NCHUNK):
        pltpu.async_copy(h_hbm.at[srcbuf.at[r]], rows.at[0], gsem).wait()
        pltpu.sync_copy(rows.at[0], acc.at[dstbuf.at[r]], add=True)
    plsc.subcore_barrier()
    pltpu.sync_copy(acc.at[pl.ds(s * _RPT, _RPT)],
                    out_hbm.at[c].at[pl.ds(s * _RPT, _RPT)])


def _mm_body(x_ref, w1_ref, u_ref):
    u_ref[...] = jnp.dot(x_ref[...], w1_ref[...],
                         preferred_element_type=jnp.float32)


def _scale_body(degp_ref, u_ref, h1_ref, dinv_ref):
    deg = degp_ref[0, :, 0:1] + degp_ref[1, :, 0:1] - 1.0
    dinv = lax.rsqrt(deg)
    h1_ref[...] = u_ref[...] * dinv
    dinv_ref[...] = dinv


def _l2_body(p_ref, h1_ref, dinv_ref, b1_ref, w2_ref, h2_ref):
    agg = p_ref[0] + p_ref[1] - h1_ref[...]
    t = jnp.maximum(agg * dinv_ref[...] + b1_ref[...], 0.0)
    h2 = jnp.dot(t, w2_ref[...], preferred_element_type=jnp.float32)
    h2_ref[...] = h2 * dinv_ref[...]


def _out_body(p_ref, h2_ref, dinv_ref, b2_ref, o_ref):
    agg = p_ref[0] + p_ref[1] - h2_ref[...]
    o = agg * dinv_ref[...] + b2_ref[...]
    m = jnp.max(o, axis=1, keepdims=True)
    lse = jnp.log(jnp.sum(jnp.exp(o - m), axis=1, keepdims=True))
    o_ref[...] = o - m - lse


def _row_spec(cols):
    return pl.BlockSpec((_R, cols), lambda i: (i, 0))


def _full_spec(shape):
    return pl.BlockSpec(shape, lambda i: tuple(0 for _ in shape))


def kernel(x, edge_index, W1, b1, W2, b2):
    src = edge_index[0].reshape(_NW, _NCHUNK, _CH)
    dst = edge_index[1].reshape(_NW, _NCHUNK, _CH)
    ones8 = jnp.ones((_N, _DW), jnp.float32)

    deg_parts = _deg_kernel(dst, ones8)

    u = pl.pallas_call(
        _mm_body,
        grid=(_GRID,),
        in_specs=[_row_spec(_D), _full_spec((_D, _D))],
        out_specs=_row_spec(_D),
        out_shape=jax.ShapeDtypeStruct((_N, _D), jnp.float32),
    )(x, W1)

    h1, dinv = pl.pallas_call(
        _scale_body,
        grid=(_GRID,),
        in_specs=[pl.BlockSpec((_NC, _R, _DW), lambda i: (0, i, 0)),
                  _row_spec(_D)],
        out_specs=[_row_spec(_D), _row_spec(1)],
        out_shape=[
            jax.ShapeDtypeStruct((_N, _D), jnp.float32),
            jax.ShapeDtypeStruct((_N, 1), jnp.float32),
        ],
    )(deg_parts, u)

    p1 = _agg_kernel(h1, src, dst)

    h2 = pl.pallas_call(
        _l2_body,
        grid=(_GRID,),
        in_specs=[
            pl.BlockSpec((_NC, _R, _D), lambda i: (0, i, 0)),
            _row_spec(_D), _row_spec(1), _full_spec((1, _D)),
            _full_spec((_D, _D)),
        ],
        out_specs=_row_spec(_D),
        out_shape=jax.ShapeDtypeStruct((_N, _D), jnp.float32),
    )(p1, h1, dinv, b1.reshape(1, _D), W2)

    p2 = _agg_kernel(h2, src, dst)

    out = pl.pallas_call(
        _out_body,
        grid=(_GRID,),
        in_specs=[
            pl.BlockSpec((_NC, _R, _D), lambda i: (0, i, 0)),
            _row_spec(_D), _row_spec(1), _full_spec((1, _D)),
        ],
        out_specs=_row_spec(_D),
        out_shape=jax.ShapeDtypeStruct((_N, _D), jnp.float32),
    )(p2, h2, dinv, b2.reshape(1, _D))

    return out


# speedup vs baseline: 1.1148x; 1.0323x over previous
import functools

import jax
import jax.numpy as jnp
from jax import lax
from jax.experimental import pallas as pl
from jax.experimental.pallas import tpu as pltpu
from jax.experimental.pallas import tpu_sc as plsc

_N = 10000
_E = 320000
_D = 128

_NC = 2
_NS = 16
_NW = _NC * _NS
_EPW = _E // _NW
_CH = 80
_NCHUNK = _EPW // _CH
_K = 2
_RPT = _N // _NS

_R = 1000
_GRID = _N // _R

_mesh = plsc.VectorSubcoreMesh(core_axis_name="c", subcore_axis_name="s")


_DW = 8


@functools.partial(
    pl.kernel,
    out_type=jax.ShapeDtypeStruct((_NC, _N, _DW), jnp.float32),
    mesh=_mesh,
    scratch_types=[
        pltpu.VMEM((_NCHUNK, _CH), jnp.int32),
        pltpu.VMEM((_CH, _DW), jnp.float32),
        pltpu.VMEM_SHARED((_N, _DW), jnp.float32),
    ],
    compiler_params=pltpu.CompilerParams(use_tc_tiling_on_sc=False),
)
def _deg_kernel(ei_hbm, ones_hbm, out_hbm, dstbuf, onesbuf, acc):
    c = lax.axis_index("c")
    s = lax.axis_index("s")
    w = s * _NC + c
    pltpu.sync_copy(ei_hbm.at[1].at[w], dstbuf)
    pltpu.sync_copy(ones_hbm.at[pl.ds(0, _CH)], onesbuf)
    pltpu.sync_copy(ones_hbm.at[pl.ds(s * _RPT, _RPT)],
                    acc.at[pl.ds(s * _RPT, _RPT)])
    plsc.subcore_barrier()

    def body(j, carry):
        pltpu.sync_copy(onesbuf, acc.at[dstbuf.at[j]], add=True)
        return carry

    lax.fori_loop(0, _NCHUNK, body, 0)
    plsc.subcore_barrier()
    pltpu.sync_copy(acc.at[pl.ds(s * _RPT, _RPT)],
                    out_hbm.at[c].at[pl.ds(s * _RPT, _RPT)])


@functools.partial(
    pl.kernel,
    out_type=jax.ShapeDtypeStruct((_NC, _N, _D), jnp.float32),
    mesh=_mesh,
    scratch_types=[
        pltpu.VMEM((_NCHUNK, _CH), jnp.int32),
        pltpu.VMEM((_NCHUNK, _CH), jnp.int32),
        pltpu.VMEM((_K, _CH, _D), jnp.float32),
        pltpu.VMEM_SHARED((_N, _D), jnp.float32),
        pltpu.SemaphoreType.DMA,
    ],
    compiler_params=pltpu.CompilerParams(use_tc_tiling_on_sc=False),
)
def _agg_kernel(h_hbm, ei_hbm, out_hbm, srcbuf, dstbuf, rows, acc,
                gsem):
    c = lax.axis_index("c")
    s = lax.axis_index("s")
    w = s * _NC + c
    cp0 = pltpu.async_copy(ei_hbm.at[0].at[w], srcbuf, gsem)
    cp1 = pltpu.async_copy(ei_hbm.at[1].at[w], dstbuf, gsem)
    cp2 = pltpu.async_copy(h_hbm.at[pl.ds(s * _RPT, _RPT)],
                           acc.at[pl.ds(s * _RPT, _RPT)], gsem)
    cp0.wait()
    cp1.wait()
    cp2.wait()
    plsc.subcore_barrier()

    def group(g, carry):
        base = g * _K
        copies = [
            pltpu.async_copy(h_hbm.at[srcbuf.at[base + b]], rows.at[b], gsem)
            for b in range(_K)
        ]
        for b in range(_K):
            copies[b].wait()
            pltpu.sync_copy(rows.at[b], acc.at[dstbuf.at[base + b]], add=True)
        return carry

    lax.fori_loop(0, _NCHUNK // _K, group, 0)
    for r in range((_NCHUNK // _K) * _K, _NCHUNK):
        pltpu.async_copy(h_hbm.at[srcbuf.at[r]], rows.at[0], gsem).wait()
        pltpu.sync_copy(rows.at[0], acc.at[dstbuf.at[r]], add=True)
    plsc.subcore_barrier()
    pltpu.sync_copy(acc.at[pl.ds(s * _RPT, _RPT)],
                    out_hbm.at[c].at[pl.ds(s * _RPT, _RPT)])


def _mm_body(x_ref, w1_ref, u_ref):
    u_ref[...] = jnp.dot(x_ref[...], w1_ref[...],
                         preferred_element_type=jnp.float32)


def _scale_body(degp_ref, u_ref, h1_ref, dinv_ref):
    deg = degp_ref[0, :, 0:1] + degp_ref[1, :, 0:1] - 1.0
    dinv = lax.rsqrt(deg)
    h1_ref[...] = u_ref[...] * dinv
    dinv_ref[...] = dinv


def _l2_body(p_ref, h1_ref, dinv_ref, b1_ref, w2_ref, h2_ref):
    agg = p_ref[0] + p_ref[1] - h1_ref[...]
    t = jnp.maximum(agg * dinv_ref[...] + b1_ref[...], 0.0)
    h2 = jnp.dot(t, w2_ref[...], preferred_element_type=jnp.float32)
    h2_ref[...] = h2 * dinv_ref[...]


def _out_body(p_ref, h2_ref, dinv_ref, b2_ref, o_ref):
    agg = p_ref[0] + p_ref[1] - h2_ref[...]
    o = agg * dinv_ref[...] + b2_ref[...]
    m = jnp.max(o, axis=1, keepdims=True)
    lse = jnp.log(jnp.sum(jnp.exp(o - m), axis=1, keepdims=True))
    o_ref[...] = o - m - lse


def _row_spec(cols):
    return pl.BlockSpec((_R, cols), lambda i: (i, 0))


def _full_spec(shape):
    return pl.BlockSpec(shape, lambda i: tuple(0 for _ in shape))


def kernel(x, edge_index, W1, b1, W2, b2):
    ei4 = edge_index.reshape(2, _NW, _NCHUNK, _CH)
    ones8 = jnp.ones((_N, _DW), jnp.float32)

    deg_parts = _deg_kernel(ei4, ones8)

    u = pl.pallas_call(
        _mm_body,
        grid=(_GRID,),
        in_specs=[_row_spec(_D), _full_spec((_D, _D))],
        out_specs=_row_spec(_D),
        out_shape=jax.ShapeDtypeStruct((_N, _D), jnp.float32),
    )(x, W1)

    h1, dinv = pl.pallas_call(
        _scale_body,
        grid=(_GRID,),
        in_specs=[pl.BlockSpec((_NC, _R, _DW), lambda i: (0, i, 0)),
                  _row_spec(_D)],
        out_specs=[_row_spec(_D), _row_spec(1)],
        out_shape=[
            jax.ShapeDtypeStruct((_N, _D), jnp.float32),
            jax.ShapeDtypeStruct((_N, 1), jnp.float32),
        ],
    )(deg_parts, u)

    p1 = _agg_kernel(h1, ei4)

    h2 = pl.pallas_call(
        _l2_body,
        grid=(_GRID,),
        in_specs=[
            pl.BlockSpec((_NC, _R, _D), lambda i: (0, i, 0)),
            _row_spec(_D), _row_spec(1), _full_spec((1, _D)),
            _full_spec((_D, _D)),
        ],
        out_specs=_row_spec(_D),
        out_shape=jax.ShapeDtypeStruct((_N, _D), jnp.float32),
    )(p1, h1, dinv, b1.reshape(1, _D), W2)

    p2 = _agg_kernel(h2, ei4)

    out = pl.pallas_call(
        _out_body,
        grid=(_GRID,),
        in_specs=[
            pl.BlockSpec((_NC, _R, _D), lambda i: (0, i, 0)),
            _row_spec(_D), _row_spec(1), _full_spec((1, _D)),
        ],
        out_specs=_row_spec(_D),
        out_shape=jax.ShapeDtypeStruct((_N, _D), jnp.float32),
    )(p2, h2, dinv, b2.reshape(1, _D))

    return out


# speedup vs baseline: 1.1316x; 1.0150x over previous
import functools

import jax
import jax.numpy as jnp
from jax import lax
from jax.experimental import pallas as pl
from jax.experimental.pallas import tpu as pltpu
from jax.experimental.pallas import tpu_sc as plsc

_N = 10000
_E = 320000
_D = 128

_NC = 2
_NS = 16
_NW = _NC * _NS
_EPW = _E // _NW
_CH = 80
_NCHUNK = _EPW // _CH
_K = 2
_RPT = _N // _NS

_R = 2000
_GRID = _N // _R

_mesh = plsc.VectorSubcoreMesh(core_axis_name="c", subcore_axis_name="s")


_DW = 8


@functools.partial(
    pl.kernel,
    out_type=jax.ShapeDtypeStruct((_NC, _N, _DW), jnp.float32),
    mesh=_mesh,
    scratch_types=[
        pltpu.VMEM((_NCHUNK, _CH), jnp.int32),
        pltpu.VMEM((_CH, _DW), jnp.float32),
        pltpu.VMEM_SHARED((_N, _DW), jnp.float32),
    ],
    compiler_params=pltpu.CompilerParams(use_tc_tiling_on_sc=False),
)
def _deg_kernel(ei_hbm, ones_hbm, out_hbm, dstbuf, onesbuf, acc):
    c = lax.axis_index("c")
    s = lax.axis_index("s")
    w = s * _NC + c
    pltpu.sync_copy(ei_hbm.at[1].at[w], dstbuf)
    pltpu.sync_copy(ones_hbm.at[pl.ds(0, _CH)], onesbuf)
    pltpu.sync_copy(ones_hbm.at[pl.ds(s * _RPT, _RPT)],
                    acc.at[pl.ds(s * _RPT, _RPT)])
    plsc.subcore_barrier()

    def body(j, carry):
        pltpu.sync_copy(onesbuf, acc.at[dstbuf.at[j]], add=True)
        return carry

    lax.fori_loop(0, _NCHUNK, body, 0)
    plsc.subcore_barrier()
    pltpu.sync_copy(acc.at[pl.ds(s * _RPT, _RPT)],
                    out_hbm.at[c].at[pl.ds(s * _RPT, _RPT)])


@functools.partial(
    pl.kernel,
    out_type=jax.ShapeDtypeStruct((_NC, _N, _D), jnp.float32),
    mesh=_mesh,
    scratch_types=[
        pltpu.VMEM((_NCHUNK, _CH), jnp.int32),
        pltpu.VMEM((_NCHUNK, _CH), jnp.int32),
        pltpu.VMEM((_K, _CH, _D), jnp.float32),
        pltpu.VMEM_SHARED((_N, _D), jnp.float32),
        pltpu.SemaphoreType.DMA,
    ],
    compiler_params=pltpu.CompilerParams(use_tc_tiling_on_sc=False),
)
def _agg_kernel(h_hbm, ei_hbm, out_hbm, srcbuf, dstbuf, rows, acc,
                gsem):
    c = lax.axis_index("c")
    s = lax.axis_index("s")
    w = s * _NC + c
    cp0 = pltpu.async_copy(ei_hbm.at[0].at[w], srcbuf, gsem)
    cp1 = pltpu.async_copy(ei_hbm.at[1].at[w], dstbuf, gsem)
    cp2 = pltpu.async_copy(h_hbm.at[pl.ds(s * _RPT, _RPT)],
                           acc.at[pl.ds(s * _RPT, _RPT)], gsem)
    cp0.wait()
    cp1.wait()
    cp2.wait()
    plsc.subcore_barrier()

    def group(g, carry):
        base = g * _K
        copies = [
            pltpu.async_copy(h_hbm.at[srcbuf.at[base + b]], rows.at[b], gsem)
            for b in range(_K)
        ]
        for b in range(_K):
            copies[b].wait()
            pltpu.sync_copy(rows.at[b], acc.at[dstbuf.at[base + b]], add=True)
        return carry

    lax.fori_loop(0, _NCHUNK // _K, group, 0)
    for r in range((_NCHUNK // _K) * _K, _NCHUNK):
        pltpu.async_copy(h_hbm.at[srcbuf.at[r]], rows.at[0], gsem).wait()
        pltpu.sync_copy(rows.at[0], acc.at[dstbuf.at[r]], add=True)
    plsc.subcore_barrier()
    pltpu.sync_copy(acc.at[pl.ds(s * _RPT, _RPT)],
                    out_hbm.at[c].at[pl.ds(s * _RPT, _RPT)])


def _mm_body(x_ref, w1_ref, u_ref):
    u_ref[...] = jnp.dot(x_ref[...], w1_ref[...],
                         preferred_element_type=jnp.float32)


def _scale_body(degp_ref, u_ref, h1_ref, dinv_ref):
    deg = degp_ref[0, :, 0:1] + degp_ref[1, :, 0:1] - 1.0
    dinv = lax.rsqrt(deg)
    h1_ref[...] = u_ref[...] * dinv
    dinv_ref[...] = dinv


def _l2_body(p_ref, h1_ref, dinv_ref, b1_ref, w2_ref, h2_ref):
    agg = p_ref[0] + p_ref[1] - h1_ref[...]
    t = jnp.maximum(agg * dinv_ref[...] + b1_ref[...], 0.0)
    h2 = jnp.dot(t, w2_ref[...], preferred_element_type=jnp.float32)
    h2_ref[...] = h2 * dinv_ref[...]


def _out_body(p_ref, h2_ref, dinv_ref, b2_ref, o_ref):
    agg = p_ref[0] + p_ref[1] - h2_ref[...]
    o = agg * dinv_ref[...] + b2_ref[...]
    m = jnp.max(o, axis=1, keepdims=True)
    lse = jnp.log(jnp.sum(jnp.exp(o - m), axis=1, keepdims=True))
    o_ref[...] = o - m - lse


def _row_spec(cols):
    return pl.BlockSpec((_R, cols), lambda i: (i, 0))


def _full_spec(shape):
    return pl.BlockSpec(shape, lambda i: tuple(0 for _ in shape))


def kernel(x, edge_index, W1, b1, W2, b2):
    ei4 = edge_index.reshape(2, _NW, _NCHUNK, _CH)
    ones8 = jnp.ones((_N, _DW), jnp.float32)

    deg_parts = _deg_kernel(ei4, ones8)

    u = pl.pallas_call(
        _mm_body,
        grid=(_GRID,),
        in_specs=[_row_spec(_D), _full_spec((_D, _D))],
        out_specs=_row_spec(_D),
        out_shape=jax.ShapeDtypeStruct((_N, _D), jnp.float32),
    )(x, W1)

    h1, dinv = pl.pallas_call(
        _scale_body,
        grid=(_GRID,),
        in_specs=[pl.BlockSpec((_NC, _R, _DW), lambda i: (0, i, 0)),
                  _row_spec(_D)],
        out_specs=[_row_spec(_D), _row_spec(1)],
        out_shape=[
            jax.ShapeDtypeStruct((_N, _D), jnp.float32),
            jax.ShapeDtypeStruct((_N, 1), jnp.float32),
        ],
    )(deg_parts, u)

    p1 = _agg_kernel(h1, ei4)

    h2 = pl.pallas_call(
        _l2_body,
        grid=(_GRID,),
        in_specs=[
            pl.BlockSpec((_NC, _R, _D), lambda i: (0, i, 0)),
            _row_spec(_D), _row_spec(1), _full_spec((1, _D)),
            _full_spec((_D, _D)),
        ],
        out_specs=_row_spec(_D),
        out_shape=jax.ShapeDtypeStruct((_N, _D), jnp.float32),
    )(p1, h1, dinv, b1.reshape(1, _D), W2)

    p2 = _agg_kernel(h2, ei4)

    out = pl.pallas_call(
        _out_body,
        grid=(_GRID,),
        in_specs=[
            pl.BlockSpec((_NC, _R, _D), lambda i: (0, i, 0)),
            _row_spec(_D), _row_spec(1), _full_spec((1, _D)),
        ],
        out_specs=_row_spec(_D),
        out_shape=jax.ShapeDtypeStruct((_N, _D), jnp.float32),
    )(p2, h2, dinv, b2.reshape(1, _D))

    return out


# speedup vs baseline: 1.5369x; 1.3582x over previous
import functools

import jax
import jax.numpy as jnp
from jax import lax
from jax.experimental import pallas as pl
from jax.experimental.pallas import tpu as pltpu
from jax.experimental.pallas import tpu_sc as plsc

_N = 10000
_E = 320000
_D = 128

_NC = 2
_NS = 16
_NW = _NC * _NS
_EPW = _E // _NW
_CH = 80
_NCHUNK = _EPW // _CH
_K = 3
_RPT = _N // _NS

_R = 2000
_GRID = _N // _R

_mesh = plsc.VectorSubcoreMesh(core_axis_name="c", subcore_axis_name="s")


_DW = 8


@functools.partial(
    pl.kernel,
    out_type=jax.ShapeDtypeStruct((_NC, _N, _DW), jnp.float32),
    mesh=_mesh,
    scratch_types=[
        pltpu.VMEM((_NCHUNK, _CH), jnp.int32),
        pltpu.VMEM((_CH, _DW), jnp.float32),
        pltpu.VMEM_SHARED((_N, _DW), jnp.float32),
    ],
    compiler_params=pltpu.CompilerParams(use_tc_tiling_on_sc=False),
)
def _deg_kernel(ei_hbm, ones_hbm, out_hbm, dstbuf, onesbuf, acc):
    c = lax.axis_index("c")
    s = lax.axis_index("s")
    w = s * _NC + c
    pltpu.sync_copy(ei_hbm.at[1].at[w], dstbuf)
    pltpu.sync_copy(ones_hbm.at[pl.ds(0, _CH)], onesbuf)
    pltpu.sync_copy(ones_hbm.at[pl.ds(s * _RPT, _RPT)],
                    acc.at[pl.ds(s * _RPT, _RPT)])
    plsc.subcore_barrier()

    def body(j, carry):
        pltpu.sync_copy(onesbuf, acc.at[dstbuf.at[j]], add=True)
        return carry

    lax.fori_loop(0, _NCHUNK, body, 0)
    plsc.subcore_barrier()
    pltpu.sync_copy(acc.at[pl.ds(s * _RPT, _RPT)],
                    out_hbm.at[c].at[pl.ds(s * _RPT, _RPT)])


@functools.partial(
    pl.kernel,
    out_type=jax.ShapeDtypeStruct((_NC, _N, _D), jnp.float32),
    mesh=_mesh,
    scratch_types=[
        pltpu.VMEM((_NCHUNK, _CH), jnp.int32),
        pltpu.VMEM((_NCHUNK, _CH), jnp.int32),
        pltpu.VMEM((_K, _CH, _D), jnp.float32),
        pltpu.VMEM_SHARED((_N, _D), jnp.float32),
        [pltpu.SemaphoreType.DMA] * _K,
    ],
    compiler_params=pltpu.CompilerParams(use_tc_tiling_on_sc=False),
)
def _agg_kernel(h_hbm, ei_hbm, out_hbm, srcbuf, dstbuf, rows, acc,
                gsems):
    c = lax.axis_index("c")
    s = lax.axis_index("s")
    w = s * _NC + c
    cp0 = pltpu.async_copy(ei_hbm.at[0].at[w], srcbuf, gsems[0])
    cp1 = pltpu.async_copy(ei_hbm.at[1].at[w], dstbuf, gsems[1])
    cp2 = pltpu.async_copy(h_hbm.at[pl.ds(s * _RPT, _RPT)],
                           acc.at[pl.ds(s * _RPT, _RPT)], gsems[2])
    cp0.wait()
    cp1.wait()
    cp2.wait()
    plsc.subcore_barrier()

    def fire_g(chunk, b):
        pltpu.async_copy(h_hbm.at[srcbuf.at[chunk]], rows.at[b], gsems[b])

    def wait_g(b):
        pltpu.make_async_copy(h_hbm.at[srcbuf.at[0]], rows.at[b],
                              gsems[b]).wait()

    fire_g(0, 0)
    fire_g(1, 1)

    def group(g, carry):
        base = g * _K
        for b in range(_K):
            j = base + b
            fire_g(j + 2, (b + 2) % _K)
            wait_g(b)
            pltpu.sync_copy(rows.at[b], acc.at[dstbuf.at[j]], add=True)
        return carry

    lax.fori_loop(0, (_NCHUNK - 2) // _K, group, 0)
    for j in range(_NCHUNK - 2, _NCHUNK):
        b = j % _K
        wait_g(b)
        pltpu.sync_copy(rows.at[b], acc.at[dstbuf.at[j]], add=True)
    plsc.subcore_barrier()
    pltpu.sync_copy(acc.at[pl.ds(s * _RPT, _RPT)],
                    out_hbm.at[c].at[pl.ds(s * _RPT, _RPT)])


def _mm_body(x_ref, w1_ref, u_ref):
    u_ref[...] = jnp.dot(x_ref[...], w1_ref[...],
                         preferred_element_type=jnp.float32)


def _scale_body(degp_ref, u_ref, h1_ref, dinv_ref):
    deg = degp_ref[0, :, 0:1] + degp_ref[1, :, 0:1] - 1.0
    dinv = lax.rsqrt(deg)
    h1_ref[...] = u_ref[...] * dinv
    dinv_ref[...] = dinv


def _l2_body(p_ref, h1_ref, dinv_ref, b1_ref, w2_ref, h2_ref):
    agg = p_ref[0] + p_ref[1] - h1_ref[...]
    t = jnp.maximum(agg * dinv_ref[...] + b1_ref[...], 0.0)
    h2 = jnp.dot(t, w2_ref[...], preferred_element_type=jnp.float32)
    h2_ref[...] = h2 * dinv_ref[...]


def _out_body(p_ref, h2_ref, dinv_ref, b2_ref, o_ref):
    agg = p_ref[0] + p_ref[1] - h2_ref[...]
    o = agg * dinv_ref[...] + b2_ref[...]
    m = jnp.max(o, axis=1, keepdims=True)
    lse = jnp.log(jnp.sum(jnp.exp(o - m), axis=1, keepdims=True))
    o_ref[...] = o - m - lse


def _row_spec(cols):
    return pl.BlockSpec((_R, cols), lambda i: (i, 0))


def _full_spec(shape):
    return pl.BlockSpec(shape, lambda i: tuple(0 for _ in shape))


def kernel(x, edge_index, W1, b1, W2, b2):
    ei4 = edge_index.reshape(2, _NW, _NCHUNK, _CH)
    ones8 = jnp.ones((_N, _DW), jnp.float32)

    deg_parts = _deg_kernel(ei4, ones8)

    u = pl.pallas_call(
        _mm_body,
        grid=(_GRID,),
        in_specs=[_row_spec(_D), _full_spec((_D, _D))],
        out_specs=_row_spec(_D),
        out_shape=jax.ShapeDtypeStruct((_N, _D), jnp.float32),
    )(x, W1)

    h1, dinv = pl.pallas_call(
        _scale_body,
        grid=(_GRID,),
        in_specs=[pl.BlockSpec((_NC, _R, _DW), lambda i: (0, i, 0)),
                  _row_spec(_D)],
        out_specs=[_row_spec(_D), _row_spec(1)],
        out_shape=[
            jax.ShapeDtypeStruct((_N, _D), jnp.float32),
            jax.ShapeDtypeStruct((_N, 1), jnp.float32),
        ],
    )(deg_parts, u)

    p1 = _agg_kernel(h1, ei4)

    h2 = pl.pallas_call(
        _l2_body,
        grid=(_GRID,),
        in_specs=[
            pl.BlockSpec((_NC, _R, _D), lambda i: (0, i, 0)),
            _row_spec(_D), _row_spec(1), _full_spec((1, _D)),
            _full_spec((_D, _D)),
        ],
        out_specs=_row_spec(_D),
        out_shape=jax.ShapeDtypeStruct((_N, _D), jnp.float32),
    )(p1, h1, dinv, b1.reshape(1, _D), W2)

    p2 = _agg_kernel(h2, ei4)

    out = pl.pallas_call(
        _out_body,
        grid=(_GRID,),
        in_specs=[
            pl.BlockSpec((_NC, _R, _D), lambda i: (0, i, 0)),
            _row_spec(_D), _row_spec(1), _full_spec((1, _D)),
        ],
        out_specs=_row_spec(_D),
        out_shape=jax.ShapeDtypeStruct((_N, _D), jnp.float32),
    )(p2, h2, dinv, b2.reshape(1, _D))

    return out


# speedup vs baseline: 1.6168x; 1.0520x over previous
import functools

import jax
import jax.numpy as jnp
from jax import lax
from jax.experimental import pallas as pl
from jax.experimental.pallas import tpu as pltpu
from jax.experimental.pallas import tpu_sc as plsc

_N = 10000
_E = 320000
_D = 128

_NC = 2
_NS = 16
_NW = _NC * _NS
_EPW = _E // _NW
_CH = 80
_NCHUNK = _EPW // _CH
_K = 3
_RPT = _N // _NS

_R = 2000
_GRID = _N // _R

_mesh = plsc.VectorSubcoreMesh(core_axis_name="c", subcore_axis_name="s")


_DW = 8


@functools.partial(
    pl.kernel,
    out_type=jax.ShapeDtypeStruct((_NC, _N, _D), jnp.float32),
    mesh=_mesh,
    scratch_types=[
        pltpu.VMEM((_NCHUNK, _CH), jnp.int32),
        pltpu.VMEM((_CH, _DW), jnp.float32),
        pltpu.VMEM_SHARED((_N, _DW), jnp.float32),
        pltpu.SemaphoreType.DMA,
    ],
    compiler_params=pltpu.CompilerParams(use_tc_tiling_on_sc=False),
)
def _deg_kernel(ei_hbm, ones_hbm, out_hbm, dstbuf, onesbuf, acc, ssem):
    c = lax.axis_index("c")
    s = lax.axis_index("s")
    w = s * _NC + c
    cp0 = pltpu.async_copy(ei_hbm.at[1].at[w], dstbuf, ssem)
    cp1 = pltpu.async_copy(ones_hbm.at[pl.ds(0, _CH)], onesbuf, ssem)
    cp2 = pltpu.async_copy(ones_hbm.at[pl.ds(s * _RPT, _RPT)],
                           acc.at[pl.ds(s * _RPT, _RPT)], ssem)
    cp0.wait()
    cp1.wait()
    cp2.wait()
    plsc.subcore_barrier()

    def body(g, carry):
        base = g * 5
        copies = [
            pltpu.async_copy(onesbuf, acc.at[dstbuf.at[base + k]], ssem,
                             add=True)
            for k in range(5)
        ]
        for cp in copies:
            cp.wait()
        return carry

    lax.fori_loop(0, _NCHUNK // 5, body, 0)
    plsc.subcore_barrier()
    pltpu.sync_copy(acc.at[pl.ds(s * _RPT, _RPT)],
                    out_hbm.at[c].at[pl.ds(s * _RPT, _RPT), pl.ds(0, _DW)])


@functools.partial(
    pl.kernel,
    out_type=jax.ShapeDtypeStruct((_NC, _N, _D), jnp.float32),
    mesh=_mesh,
    scratch_types=[
        pltpu.VMEM((_NCHUNK, _CH), jnp.int32),
        pltpu.VMEM((_NCHUNK, _CH), jnp.int32),
        pltpu.VMEM((_K, _CH, _D), jnp.float32),
        pltpu.VMEM_SHARED((_N, _D), jnp.float32),
        [pltpu.SemaphoreType.DMA] * _K,
    ],
    compiler_params=pltpu.CompilerParams(use_tc_tiling_on_sc=False),
)
def _agg_kernel(h_hbm, ei_hbm, out_hbm, srcbuf, dstbuf, rows, acc,
                gsems):
    c = lax.axis_index("c")
    s = lax.axis_index("s")
    w = s * _NC + c
    cp0 = pltpu.async_copy(ei_hbm.at[0].at[w], srcbuf, gsems[0])
    cp1 = pltpu.async_copy(ei_hbm.at[1].at[w], dstbuf, gsems[1])
    cp2 = pltpu.async_copy(h_hbm.at[pl.ds(s * _RPT, _RPT)],
                           acc.at[pl.ds(s * _RPT, _RPT)], gsems[2])
    cp0.wait()
    cp1.wait()
    cp2.wait()
    plsc.subcore_barrier()

    def fire_g(chunk, b):
        pltpu.async_copy(h_hbm.at[srcbuf.at[chunk]], rows.at[b], gsems[b])

    def wait_g(b):
        pltpu.make_async_copy(h_hbm.at[srcbuf.at[0]], rows.at[b],
                              gsems[b]).wait()

    fire_g(0, 0)
    fire_g(1, 1)

    def group(g, carry):
        base = g * _K
        for b in range(_K):
            j = base + b
            fire_g(j + 2, (b + 2) % _K)
            wait_g(b)
            pltpu.sync_copy(rows.at[b], acc.at[dstbuf.at[j]], add=True)
        return carry

    lax.fori_loop(0, (_NCHUNK - 2) // _K, group, 0)
    for j in range(_NCHUNK - 2, _NCHUNK):
        b = j % _K
        wait_g(b)
        pltpu.sync_copy(rows.at[b], acc.at[dstbuf.at[j]], add=True)
    plsc.subcore_barrier()
    pltpu.sync_copy(acc.at[pl.ds(s * _RPT, _RPT)],
                    out_hbm.at[c].at[pl.ds(s * _RPT, _RPT)])


def _mm_body(x_ref, w1_ref, u_ref):
    u_ref[...] = jnp.dot(x_ref[...], w1_ref[...],
                         preferred_element_type=jnp.float32)


def _scale_body(degp_ref, u_ref, h1_ref, dinv_ref):
    deg = degp_ref[0, :, 0:1] + degp_ref[1, :, 0:1] - 1.0
    dinv = lax.rsqrt(deg)
    h1_ref[...] = u_ref[...] * dinv
    dinv_ref[...] = dinv


def _l2_body(p_ref, h1_ref, dinv_ref, b1_ref, w2_ref, h2_ref):
    agg = p_ref[0] + p_ref[1] - h1_ref[...]
    t = jnp.maximum(agg * dinv_ref[...] + b1_ref[...], 0.0)
    h2 = jnp.dot(t, w2_ref[...], preferred_element_type=jnp.float32)
    h2_ref[...] = h2 * dinv_ref[...]


def _out_body(p_ref, h2_ref, dinv_ref, b2_ref, o_ref):
    agg = p_ref[0] + p_ref[1] - h2_ref[...]
    o = agg * dinv_ref[...] + b2_ref[...]
    m = jnp.max(o, axis=1, keepdims=True)
    lse = jnp.log(jnp.sum(jnp.exp(o - m), axis=1, keepdims=True))
    o_ref[...] = o - m - lse


def _row_spec(cols):
    return pl.BlockSpec((_R, cols), lambda i: (i, 0))


def _full_spec(shape):
    return pl.BlockSpec(shape, lambda i: tuple(0 for _ in shape))


def kernel(x, edge_index, W1, b1, W2, b2):
    ei4 = edge_index.reshape(2, _NW, _NCHUNK, _CH)
    ones8 = jnp.ones((_N, _DW), jnp.float32)

    deg_parts = _deg_kernel(ei4, ones8)

    u = pl.pallas_call(
        _mm_body,
        grid=(_GRID,),
        in_specs=[_row_spec(_D), _full_spec((_D, _D))],
        out_specs=_row_spec(_D),
        out_shape=jax.ShapeDtypeStruct((_N, _D), jnp.float32),
    )(x, W1)

    h1, dinv = pl.pallas_call(
        _scale_body,
        grid=(_GRID,),
        in_specs=[pl.BlockSpec((_NC, _R, _D), lambda i: (0, i, 0)),
                  _row_spec(_D)],
        out_specs=[_row_spec(_D), _row_spec(1)],
        out_shape=[
            jax.ShapeDtypeStruct((_N, _D), jnp.float32),
            jax.ShapeDtypeStruct((_N, 1), jnp.float32),
        ],
    )(deg_parts, u)

    p1 = _agg_kernel(h1, ei4)

    h2 = pl.pallas_call(
        _l2_body,
        grid=(_GRID,),
        in_specs=[
            pl.BlockSpec((_NC, _R, _D), lambda i: (0, i, 0)),
            _row_spec(_D), _row_spec(1), _full_spec((1, _D)),
            _full_spec((_D, _D)),
        ],
        out_specs=_row_spec(_D),
        out_shape=jax.ShapeDtypeStruct((_N, _D), jnp.float32),
    )(p1, h1, dinv, b1.reshape(1, _D), W2)

    p2 = _agg_kernel(h2, ei4)

    out = pl.pallas_call(
        _out_body,
        grid=(_GRID,),
        in_specs=[
            pl.BlockSpec((_NC, _R, _D), lambda i: (0, i, 0)),
            _row_spec(_D), _row_spec(1), _full_spec((1, _D)),
        ],
        out_specs=_row_spec(_D),
        out_shape=jax.ShapeDtypeStruct((_N, _D), jnp.float32),
    )(p2, h2, dinv, b2.reshape(1, _D))

    return out


# speedup vs baseline: 1.6701x; 1.0329x over previous
import functools

import jax
import jax.numpy as jnp
from jax import lax
from jax.experimental import pallas as pl
from jax.experimental.pallas import tpu as pltpu
from jax.experimental.pallas import tpu_sc as plsc

_N = 10000
_E = 320000
_D = 128

_NC = 2
_NS = 16
_NW = _NC * _NS
_EPW = _E // _NW
_CH = 80
_NCHUNK = _EPW // _CH
_K = 3
_RPT = _N // _NS

_R = 2000
_GRID = _N // _R

_mesh = plsc.VectorSubcoreMesh(core_axis_name="c", subcore_axis_name="s")


_DW = 8


@functools.partial(
    pl.kernel,
    out_type=jax.ShapeDtypeStruct((_NC, _N, _D), jnp.float32),
    mesh=_mesh,
    scratch_types=[
        pltpu.VMEM((_NCHUNK, _CH), jnp.int32),
        pltpu.VMEM((_CH, _DW), jnp.float32),
        pltpu.VMEM_SHARED((_N, _DW), jnp.float32),
        pltpu.SemaphoreType.DMA,
    ],
    compiler_params=pltpu.CompilerParams(use_tc_tiling_on_sc=False),
)
def _deg_kernel(ei_hbm, ones_hbm, out_hbm, dstbuf, onesbuf, acc, ssem):
    c = lax.axis_index("c")
    s = lax.axis_index("s")
    w = s * _NC + c
    cp0 = pltpu.async_copy(ei_hbm.at[1].at[w], dstbuf, ssem)
    cp1 = pltpu.async_copy(ones_hbm.at[pl.ds(0, _CH)], onesbuf, ssem)
    cp2 = pltpu.async_copy(ones_hbm.at[pl.ds(s * _RPT, _RPT)],
                           acc.at[pl.ds(s * _RPT, _RPT)], ssem)
    cp0.wait()
    cp1.wait()
    cp2.wait()
    plsc.subcore_barrier()

    def body(g, carry):
        base = g * 5
        copies = [
            pltpu.async_copy(onesbuf, acc.at[dstbuf.at[base + k]], ssem,
                             add=True)
            for k in range(5)
        ]
        for cp in copies:
            cp.wait()
        return carry

    lax.fori_loop(0, _NCHUNK // 5, body, 0)
    plsc.subcore_barrier()
    pltpu.sync_copy(acc.at[pl.ds(s * _RPT, _RPT)],
                    out_hbm.at[c].at[pl.ds(s * _RPT, _RPT), pl.ds(0, _DW)])


@functools.partial(
    pl.kernel,
    out_type=jax.ShapeDtypeStruct((_NC, _N, _D), jnp.bfloat16),
    mesh=_mesh,
    scratch_types=[
        pltpu.VMEM((_NCHUNK, _CH), jnp.int32),
        pltpu.VMEM((_NCHUNK, _CH), jnp.int32),
        pltpu.VMEM((_K, _CH, _D), jnp.bfloat16),
        pltpu.VMEM_SHARED((_N, _D), jnp.bfloat16),
        [pltpu.SemaphoreType.DMA] * _K,
    ],
    compiler_params=pltpu.CompilerParams(use_tc_tiling_on_sc=False),
)
def _agg_kernel(h_hbm, ei_hbm, out_hbm, srcbuf, dstbuf, rows, acc,
                gsems):
    c = lax.axis_index("c")
    s = lax.axis_index("s")
    w = s * _NC + c
    cp0 = pltpu.async_copy(ei_hbm.at[0].at[w], srcbuf, gsems[0])
    cp1 = pltpu.async_copy(ei_hbm.at[1].at[w], dstbuf, gsems[1])
    cp2 = pltpu.async_copy(h_hbm.at[pl.ds(s * _RPT, _RPT)],
                           acc.at[pl.ds(s * _RPT, _RPT)], gsems[2])
    cp0.wait()
    cp1.wait()
    cp2.wait()
    plsc.subcore_barrier()

    def fire_g(chunk, b):
        pltpu.async_copy(h_hbm.at[srcbuf.at[chunk]], rows.at[b], gsems[b])

    def wait_g(b):
        pltpu.make_async_copy(h_hbm.at[srcbuf.at[0]], rows.at[b],
                              gsems[b]).wait()

    fire_g(0, 0)
    fire_g(1, 1)

    def group(g, carry):
        base = g * _K
        for b in range(_K):
            j = base + b
            fire_g(j + 2, (b + 2) % _K)
            wait_g(b)
            pltpu.sync_copy(rows.at[b], acc.at[dstbuf.at[j]], add=True)
        return carry

    lax.fori_loop(0, (_NCHUNK - 2) // _K, group, 0)
    for j in range(_NCHUNK - 2, _NCHUNK):
        b = j % _K
        wait_g(b)
        pltpu.sync_copy(rows.at[b], acc.at[dstbuf.at[j]], add=True)
    plsc.subcore_barrier()
    pltpu.sync_copy(acc.at[pl.ds(s * _RPT, _RPT)],
                    out_hbm.at[c].at[pl.ds(s * _RPT, _RPT)])


def _mm_body(x_ref, w1_ref, u_ref):
    u_ref[...] = jnp.dot(x_ref[...], w1_ref[...],
                         preferred_element_type=jnp.float32)


def _scale_body(degp_ref, u_ref, h1_ref, dinv_ref):
    deg = degp_ref[0, :, 0:1] + degp_ref[1, :, 0:1] - 1.0
    dinv = lax.rsqrt(deg)
    h1_ref[...] = (u_ref[...] * dinv).astype(jnp.bfloat16)
    dinv_ref[...] = dinv


def _l2_body(p_ref, h1_ref, dinv_ref, b1_ref, w2_ref, h2_ref):
    agg = (p_ref[0].astype(jnp.float32) + p_ref[1].astype(jnp.float32)
           - h1_ref[...].astype(jnp.float32))
    t = jnp.maximum(agg * dinv_ref[...] + b1_ref[...], 0.0)
    h2 = jnp.dot(t, w2_ref[...], preferred_element_type=jnp.float32)
    h2_ref[...] = (h2 * dinv_ref[...]).astype(jnp.bfloat16)


def _out_body(p_ref, h2_ref, dinv_ref, b2_ref, o_ref):
    agg = (p_ref[0].astype(jnp.float32) + p_ref[1].astype(jnp.float32)
           - h2_ref[...].astype(jnp.float32))
    o = agg * dinv_ref[...] + b2_ref[...]
    m = jnp.max(o, axis=1, keepdims=True)
    lse = jnp.log(jnp.sum(jnp.exp(o - m), axis=1, keepdims=True))
    o_ref[...] = o - m - lse


def _row_spec(cols):
    return pl.BlockSpec((_R, cols), lambda i: (i, 0))


def _full_spec(shape):
    return pl.BlockSpec(shape, lambda i: tuple(0 for _ in shape))


def kernel(x, edge_index, W1, b1, W2, b2):
    ei4 = edge_index.reshape(2, _NW, _NCHUNK, _CH)
    ones8 = jnp.ones((_N, _DW), jnp.float32)

    deg_parts = _deg_kernel(ei4, ones8)

    u = pl.pallas_call(
        _mm_body,
        grid=(_GRID,),
        in_specs=[_row_spec(_D), _full_spec((_D, _D))],
        out_specs=_row_spec(_D),
        out_shape=jax.ShapeDtypeStruct((_N, _D), jnp.float32),
    )(x, W1)

    h1, dinv = pl.pallas_call(
        _scale_body,
        grid=(_GRID,),
        in_specs=[pl.BlockSpec((_NC, _R, _D), lambda i: (0, i, 0)),
                  _row_spec(_D)],
        out_specs=[_row_spec(_D), _row_spec(1)],
        out_shape=[
            jax.ShapeDtypeStruct((_N, _D), jnp.bfloat16),
            jax.ShapeDtypeStruct((_N, 1), jnp.float32),
        ],
    )(deg_parts, u)

    p1 = _agg_kernel(h1, ei4)

    h2 = pl.pallas_call(
        _l2_body,
        grid=(_GRID,),
        in_specs=[
            pl.BlockSpec((_NC, _R, _D), lambda i: (0, i, 0)),
            _row_spec(_D), _row_spec(1), _full_spec((1, _D)),
            _full_spec((_D, _D)),
        ],
        out_specs=_row_spec(_D),
        out_shape=jax.ShapeDtypeStruct((_N, _D), jnp.bfloat16),
    )(p1, h1, dinv, b1.reshape(1, _D), W2)

    p2 = _agg_kernel(h2, ei4)

    out = pl.pallas_call(
        _out_body,
        grid=(_GRID,),
        in_specs=[
            pl.BlockSpec((_NC, _R, _D), lambda i: (0, i, 0)),
            _row_spec(_D), _row_spec(1), _full_spec((1, _D)),
        ],
        out_specs=_row_spec(_D),
        out_shape=jax.ShapeDtypeStruct((_N, _D), jnp.float32),
    )(p2, h2, dinv, b2.reshape(1, _D))

    return out


# speedup vs baseline: 1.7888x; 1.0711x over previous
import functools

import jax
import jax.numpy as jnp
from jax import lax
from jax.experimental import pallas as pl
from jax.experimental.pallas import tpu as pltpu
from jax.experimental.pallas import tpu_sc as plsc

_N = 10000
_E = 320000
_D = 128

_NC = 2
_NS = 16
_NW = _NC * _NS
_EPW = _E // _NW
_CH = 80
_NCHUNK = _EPW // _CH
_K = 4
_RPT = _N // _NS

_R = 2000
_GRID = _N // _R

_mesh = plsc.VectorSubcoreMesh(core_axis_name="c", subcore_axis_name="s")


_DW = 8


@functools.partial(
    pl.kernel,
    out_type=jax.ShapeDtypeStruct((_NC, _N, _D), jnp.float32),
    mesh=_mesh,
    scratch_types=[
        pltpu.VMEM((_NCHUNK, _CH), jnp.int32),
        pltpu.VMEM((_CH, _DW), jnp.float32),
        pltpu.VMEM_SHARED((_N, _DW), jnp.float32),
        pltpu.SemaphoreType.DMA,
    ],
    compiler_params=pltpu.CompilerParams(use_tc_tiling_on_sc=False),
)
def _deg_kernel(ei_hbm, ones_hbm, out_hbm, dstbuf, onesbuf, acc, ssem):
    c = lax.axis_index("c")
    s = lax.axis_index("s")
    w = s * _NC + c
    cp0 = pltpu.async_copy(ei_hbm.at[1].at[w], dstbuf, ssem)
    cp1 = pltpu.async_copy(ones_hbm.at[pl.ds(0, _CH)], onesbuf, ssem)
    cp2 = pltpu.async_copy(ones_hbm.at[pl.ds(s * _RPT, _RPT)],
                           acc.at[pl.ds(s * _RPT, _RPT)], ssem)
    cp0.wait()
    cp1.wait()
    cp2.wait()
    plsc.subcore_barrier()

    def body(g, carry):
        base = g * 5
        copies = [
            pltpu.async_copy(onesbuf, acc.at[dstbuf.at[base + k]], ssem,
                             add=True)
            for k in range(5)
        ]
        for cp in copies:
            cp.wait()
        return carry

    lax.fori_loop(0, _NCHUNK // 5, body, 0)
    plsc.subcore_barrier()
    pltpu.sync_copy(acc.at[pl.ds(s * _RPT, _RPT)],
                    out_hbm.at[c].at[pl.ds(s * _RPT, _RPT), pl.ds(0, _DW)])


@functools.partial(
    pl.kernel,
    out_type=jax.ShapeDtypeStruct((_NC, _N, _D), jnp.bfloat16),
    mesh=_mesh,
    scratch_types=[
        pltpu.VMEM((_NCHUNK, _CH), jnp.int32),
        pltpu.VMEM((_NCHUNK, _CH), jnp.int32),
        pltpu.VMEM((_K, _CH, _D), jnp.bfloat16),
        pltpu.VMEM_SHARED((_N, _D), jnp.bfloat16),
        [pltpu.SemaphoreType.DMA] * _K,
    ],
    compiler_params=pltpu.CompilerParams(use_tc_tiling_on_sc=False),
)
def _agg_kernel(h_hbm, ei_hbm, out_hbm, srcbuf, dstbuf, rows, acc,
                gsems):
    c = lax.axis_index("c")
    s = lax.axis_index("s")
    w = s * _NC + c
    cp0 = pltpu.async_copy(ei_hbm.at[0].at[w], srcbuf, gsems[0])
    cp1 = pltpu.async_copy(ei_hbm.at[1].at[w], dstbuf, gsems[1])
    cp2 = pltpu.async_copy(h_hbm.at[pl.ds(s * _RPT, _RPT)],
                           acc.at[pl.ds(s * _RPT, _RPT)], gsems[2])
    cp0.wait()
    cp1.wait()
    cp2.wait()
    plsc.subcore_barrier()

    def fire_g(chunk, b):
        pltpu.async_copy(h_hbm.at[srcbuf.at[chunk]], rows.at[b], gsems[b])

    def wait_g(b):
        pltpu.make_async_copy(h_hbm.at[srcbuf.at[0]], rows.at[b],
                              gsems[b]).wait()

    fire_g(0, 0)
    fire_g(1, 1)
    fire_g(2, 2)

    def group(g, carry):
        base = g * _K
        for b in range(_K):
            j = base + b
            fire_g(j + 3, (b + 3) % _K)
            wait_g(b)
            pltpu.sync_copy(rows.at[b], acc.at[dstbuf.at[j]], add=True)
        return carry

    _NG = (_NCHUNK - 3) // _K
    lax.fori_loop(0, _NG, group, 0)
    for j in range(_NG * _K, _NCHUNK):
        b = j % _K
        if j + 3 < _NCHUNK:
            fire_g(j + 3, (j + 3) % _K)
        wait_g(b)
        pltpu.sync_copy(rows.at[b], acc.at[dstbuf.at[j]], add=True)
    plsc.subcore_barrier()
    pltpu.sync_copy(acc.at[pl.ds(s * _RPT, _RPT)],
                    out_hbm.at[c].at[pl.ds(s * _RPT, _RPT)])


def _mm_body(x_ref, w1_ref, u_ref):
    u_ref[...] = jnp.dot(x_ref[...], w1_ref[...],
                         preferred_element_type=jnp.float32)


def _scale_body(degp_ref, u_ref, h1_ref, dinv_ref):
    deg = degp_ref[0, :, 0:1] + degp_ref[1, :, 0:1] - 1.0
    dinv = lax.rsqrt(deg)
    h1_ref[...] = (u_ref[...] * dinv).astype(jnp.bfloat16)
    dinv_ref[...] = dinv


def _l2_body(p_ref, h1_ref, dinv_ref, b1_ref, w2_ref, h2_ref):
    agg = (p_ref[0].astype(jnp.float32) + p_ref[1].astype(jnp.float32)
           - h1_ref[...].astype(jnp.float32))
    t = jnp.maximum(agg * dinv_ref[...] + b1_ref[...], 0.0)
    h2 = jnp.dot(t, w2_ref[...], preferred_element_type=jnp.float32)
    h2_ref[...] = (h2 * dinv_ref[...]).astype(jnp.bfloat16)


def _out_body(p_ref, h2_ref, dinv_ref, b2_ref, o_ref):
    agg = (p_ref[0].astype(jnp.float32) + p_ref[1].astype(jnp.float32)
           - h2_ref[...].astype(jnp.float32))
    o = agg * dinv_ref[...] + b2_ref[...]
    m = jnp.max(o, axis=1, keepdims=True)
    lse = jnp.log(jnp.sum(jnp.exp(o - m), axis=1, keepdims=True))
    o_ref[...] = o - m - lse


def _row_spec(cols):
    return pl.BlockSpec((_R, cols), lambda i: (i, 0))


def _full_spec(shape):
    return pl.BlockSpec(shape, lambda i: tuple(0 for _ in shape))


def kernel(x, edge_index, W1, b1, W2, b2):
    ei4 = edge_index.reshape(2, _NW, _NCHUNK, _CH)
    ones8 = jnp.ones((_N, _DW), jnp.float32)

    deg_parts = _deg_kernel(ei4, ones8)

    u = pl.pallas_call(
        _mm_body,
        grid=(_GRID,),
        in_specs=[_row_spec(_D), _full_spec((_D, _D))],
        out_specs=_row_spec(_D),
        out_shape=jax.ShapeDtypeStruct((_N, _D), jnp.float32),
    )(x, W1)

    h1, dinv = pl.pallas_call(
        _scale_body,
        grid=(_GRID,),
        in_specs=[pl.BlockSpec((_NC, _R, _D), lambda i: (0, i, 0)),
                  _row_spec(_D)],
        out_specs=[_row_spec(_D), _row_spec(1)],
        out_shape=[
            jax.ShapeDtypeStruct((_N, _D), jnp.bfloat16),
            jax.ShapeDtypeStruct((_N, 1), jnp.float32),
        ],
    )(deg_parts, u)

    p1 = _agg_kernel(h1, ei4)

    h2 = pl.pallas_call(
        _l2_body,
        grid=(_GRID,),
        in_specs=[
            pl.BlockSpec((_NC, _R, _D), lambda i: (0, i, 0)),
            _row_spec(_D), _row_spec(1), _full_spec((1, _D)),
            _full_spec((_D, _D)),
        ],
        out_specs=_row_spec(_D),
        out_shape=jax.ShapeDtypeStruct((_N, _D), jnp.bfloat16),
    )(p1, h1, dinv, b1.reshape(1, _D), W2)

    p2 = _agg_kernel(h2, ei4)

    out = pl.pallas_call(
        _out_body,
        grid=(_GRID,),
        in_specs=[
            pl.BlockSpec((_NC, _R, _D), lambda i: (0, i, 0)),
            _row_spec(_D), _row_spec(1), _full_spec((1, _D)),
        ],
        out_specs=_row_spec(_D),
        out_shape=jax.ShapeDtypeStruct((_N, _D), jnp.float32),
    )(p2, h2, dinv, b2.reshape(1, _D))

    return out


# speedup vs baseline: 1.8102x; 1.0120x over previous
import functools

import jax
import jax.numpy as jnp
from jax import lax
from jax.experimental import pallas as pl
from jax.experimental.pallas import tpu as pltpu
from jax.experimental.pallas import tpu_sc as plsc

_N = 10000
_E = 320000
_D = 128

_NC = 2
_NS = 16
_NW = _NC * _NS
_EPW = _E // _NW
_CH = 80
_NCHUNK = _EPW // _CH
_K = 5
_RPT = _N // _NS

_R = 2000
_GRID = _N // _R

_mesh = plsc.VectorSubcoreMesh(core_axis_name="c", subcore_axis_name="s")


_DW = 8


@functools.partial(
    pl.kernel,
    out_type=jax.ShapeDtypeStruct((_NC, _N, _D), jnp.float32),
    mesh=_mesh,
    scratch_types=[
        pltpu.VMEM((_NCHUNK, _CH), jnp.int32),
        pltpu.VMEM((_CH, _DW), jnp.float32),
        pltpu.VMEM_SHARED((_N, _DW), jnp.float32),
        pltpu.SemaphoreType.DMA,
    ],
    compiler_params=pltpu.CompilerParams(use_tc_tiling_on_sc=False),
)
def _deg_kernel(ei_hbm, ones_hbm, out_hbm, dstbuf, onesbuf, acc, ssem):
    c = lax.axis_index("c")
    s = lax.axis_index("s")
    w = s * _NC + c
    cp0 = pltpu.async_copy(ei_hbm.at[1].at[w], dstbuf, ssem)
    cp1 = pltpu.async_copy(ones_hbm.at[pl.ds(0, _CH)], onesbuf, ssem)
    cp2 = pltpu.async_copy(ones_hbm.at[pl.ds(s * _RPT, _RPT)],
                           acc.at[pl.ds(s * _RPT, _RPT)], ssem)
    cp0.wait()
    cp1.wait()
    cp2.wait()
    plsc.subcore_barrier()

    def body(g, carry):
        base = g * 5
        copies = [
            pltpu.async_copy(onesbuf, acc.at[dstbuf.at[base + k]], ssem,
                             add=True)
            for k in range(5)
        ]
        for cp in copies:
            cp.wait()
        return carry

    lax.fori_loop(0, _NCHUNK // 5, body, 0)
    plsc.subcore_barrier()
    pltpu.sync_copy(acc.at[pl.ds(s * _RPT, _RPT)],
                    out_hbm.at[c].at[pl.ds(s * _RPT, _RPT), pl.ds(0, _DW)])


@functools.partial(
    pl.kernel,
    out_type=jax.ShapeDtypeStruct((_NC, _N, _D), jnp.bfloat16),
    mesh=_mesh,
    scratch_types=[
        pltpu.VMEM((_NCHUNK, _CH), jnp.int32),
        pltpu.VMEM((_NCHUNK, _CH), jnp.int32),
        pltpu.VMEM((_K, _CH, _D), jnp.bfloat16),
        pltpu.VMEM_SHARED((_N, _D), jnp.bfloat16),
        [pltpu.SemaphoreType.DMA] * _K,
    ],
    compiler_params=pltpu.CompilerParams(use_tc_tiling_on_sc=False),
)
def _agg_kernel(h_hbm, ei_hbm, out_hbm, srcbuf, dstbuf, rows, acc,
                gsems):
    c = lax.axis_index("c")
    s = lax.axis_index("s")
    w = s * _NC + c
    cp0 = pltpu.async_copy(ei_hbm.at[0].at[w], srcbuf, gsems[0])
    cp1 = pltpu.async_copy(ei_hbm.at[1].at[w], dstbuf, gsems[1])
    cp2 = pltpu.async_copy(h_hbm.at[pl.ds(s * _RPT, _RPT)],
                           acc.at[pl.ds(s * _RPT, _RPT)], gsems[2])
    cp0.wait()
    cp1.wait()
    cp2.wait()
    plsc.subcore_barrier()

    def fire_g(chunk, b):
        pltpu.async_copy(h_hbm.at[srcbuf.at[chunk]], rows.at[b], gsems[b])

    def wait_g(b):
        pltpu.make_async_copy(h_hbm.at[srcbuf.at[0]], rows.at[b],
                              gsems[b]).wait()

    fire_g(0, 0)
    fire_g(1, 1)
    fire_g(2, 2)
    fire_g(3, 3)

    def group(g, carry):
        base = g * _K
        for b in range(_K):
            j = base + b
            fire_g(j + 4, (b + 4) % _K)
            wait_g(b)
            pltpu.sync_copy(rows.at[b], acc.at[dstbuf.at[j]], add=True)
        return carry

    _NG = (_NCHUNK - 4) // _K
    lax.fori_loop(0, _NG, group, 0)
    for j in range(_NG * _K, _NCHUNK):
        b = j % _K
        if j + 4 < _NCHUNK:
            fire_g(j + 4, (j + 4) % _K)
        wait_g(b)
        pltpu.sync_copy(rows.at[b], acc.at[dstbuf.at[j]], add=True)
    plsc.subcore_barrier()
    pltpu.sync_copy(acc.at[pl.ds(s * _RPT, _RPT)],
                    out_hbm.at[c].at[pl.ds(s * _RPT, _RPT)])


def _mm_body(x_ref, w1_ref, u_ref):
    u_ref[...] = jnp.dot(x_ref[...], w1_ref[...],
                         preferred_element_type=jnp.float32)


def _scale_body(degp_ref, u_ref, h1_ref, dinv_ref):
    deg = degp_ref[0, :, 0:1] + degp_ref[1, :, 0:1] - 1.0
    dinv = lax.rsqrt(deg)
    h1_ref[...] = (u_ref[...] * dinv).astype(jnp.bfloat16)
    dinv_ref[...] = dinv


def _l2_body(p_ref, h1_ref, dinv_ref, b1_ref, w2_ref, h2_ref):
    agg = (p_ref[0].astype(jnp.float32) + p_ref[1].astype(jnp.float32)
           - h1_ref[...].astype(jnp.float32))
    t = jnp.maximum(agg * dinv_ref[...] + b1_ref[...], 0.0)
    h2 = jnp.dot(t, w2_ref[...], preferred_element_type=jnp.float32)
    h2_ref[...] = (h2 * dinv_ref[...]).astype(jnp.bfloat16)


def _out_body(p_ref, h2_ref, dinv_ref, b2_ref, o_ref):
    agg = (p_ref[0].astype(jnp.float32) + p_ref[1].astype(jnp.float32)
           - h2_ref[...].astype(jnp.float32))
    o = agg * dinv_ref[...] + b2_ref[...]
    m = jnp.max(o, axis=1, keepdims=True)
    lse = jnp.log(jnp.sum(jnp.exp(o - m), axis=1, keepdims=True))
    o_ref[...] = o - m - lse


def _row_spec(cols):
    return pl.BlockSpec((_R, cols), lambda i: (i, 0))


def _full_spec(shape):
    return pl.BlockSpec(shape, lambda i: tuple(0 for _ in shape))


def kernel(x, edge_index, W1, b1, W2, b2):
    ei4 = edge_index.reshape(2, _NW, _NCHUNK, _CH)
    ones8 = jnp.ones((_N, _DW), jnp.float32)

    deg_parts = _deg_kernel(ei4, ones8)

    u = pl.pallas_call(
        _mm_body,
        grid=(_GRID,),
        in_specs=[_row_spec(_D), _full_spec((_D, _D))],
        out_specs=_row_spec(_D),
        out_shape=jax.ShapeDtypeStruct((_N, _D), jnp.float32),
    )(x, W1)

    h1, dinv = pl.pallas_call(
        _scale_body,
        grid=(_GRID,),
        in_specs=[pl.BlockSpec((_NC, _R, _D), lambda i: (0, i, 0)),
                  _row_spec(_D)],
        out_specs=[_row_spec(_D), _row_spec(1)],
        out_shape=[
            jax.ShapeDtypeStruct((_N, _D), jnp.bfloat16),
            jax.ShapeDtypeStruct((_N, 1), jnp.float32),
        ],
    )(deg_parts, u)

    p1 = _agg_kernel(h1, ei4)

    h2 = pl.pallas_call(
        _l2_body,
        grid=(_GRID,),
        in_specs=[
            pl.BlockSpec((_NC, _R, _D), lambda i: (0, i, 0)),
            _row_spec(_D), _row_spec(1), _full_spec((1, _D)),
            _full_spec((_D, _D)),
        ],
        out_specs=_row_spec(_D),
        out_shape=jax.ShapeDtypeStruct((_N, _D), jnp.bfloat16),
    )(p1, h1, dinv, b1.reshape(1, _D), W2)

    p2 = _agg_kernel(h2, ei4)

    out = pl.pallas_call(
        _out_body,
        grid=(_GRID,),
        in_specs=[
            pl.BlockSpec((_NC, _R, _D), lambda i: (0, i, 0)),
            _row_spec(_D), _row_spec(1), _full_spec((1, _D)),
        ],
        out_specs=_row_spec(_D),
        out_shape=jax.ShapeDtypeStruct((_N, _D), jnp.float32),
    )(p2, h2, dinv, b2.reshape(1, _D))

    return out
